# baseline (speedup 1.0000x reference)
import functools

import jax
import jax.numpy as jnp
from jax import lax
from jax.experimental import pallas as pl
from jax.experimental.pallas import tpu as pltpu
from jax.experimental.pallas import tpu_sc as plsc

N = 100000
D = 128
C = 400
G = 80
NC = 2
NS = 16
NW = NC * NS
NUM_CHUNKS = N // C


def _body(x0_hbm, x1_hbm, wp_hbm, out_hbm, flg_v, idx_v, x1_v, out_v, sem):
  cid = lax.axis_index("c")
  sid = lax.axis_index("s")
  wid = sid * NC + cid

  def chunk_body(k, carry):
    g = k * NW + wid
    base = g * C
    pltpu.sync_copy(x0_hbm.at[pl.ds(base, C)], flg_v)
    for j in range(C // G):
      for i in range(G // 16):
        idx_v[j, pl.ds(i * 16, 16)] = flg_v[
            pl.ds(j * G + i * 16, 16)].astype(jnp.int32)
    copies = [
        pltpu.async_copy(wp_hbm.at[idx_v.at[j]],
                         out_v.at[pl.ds(j * G, G)], sem)
        for j in range(C // G)
    ]
    for c in copies:
      c.wait()
    pltpu.sync_copy(x1_hbm.at[pl.ds(g * (C // 16), C // 16), :], x1_v)
    col = jnp.full((16,), D - 1, jnp.int32)
    for i in range(C // 16):
      rows = lax.iota(jnp.int32, 16) + (i * 16)
      plsc.store_scatter(out_v, [rows, col], x1_v[i, :])
    pltpu.sync_copy(out_v, out_hbm.at[pl.ds(base, C), :])
    return carry

  nw_chunks = (NUM_CHUNKS - wid + NW - 1) // NW
  lax.fori_loop(0, nw_chunks, chunk_body, 0)


@jax.jit
def kernel(x, W):
  xt = x.T
  x0 = xt[0]
  x1 = xt[1].reshape(N // 16, 16)
  wp = jnp.pad(W, ((0, 0), (0, 1)))
  mesh = plsc.VectorSubcoreMesh(
      core_axis_name="c", subcore_axis_name="s", num_cores=NC,
      num_subcores=NS)
  run = pl.kernel(
      _body,
      out_type=jax.ShapeDtypeStruct((N, D), jnp.float32),
      mesh=mesh,
      compiler_params=pltpu.CompilerParams(use_tc_tiling_on_sc=False, needs_layout_passes=False),
      scratch_types=[
          pltpu.VMEM((C,), jnp.float32),
          pltpu.VMEM((C // G, G), jnp.int32),
          pltpu.VMEM((C // 16, 16), jnp.float32),
          pltpu.VMEM((C, D), jnp.float32),
          pltpu.SemaphoreType.DMA,
      ],
  )
  return run(x0, x1, wp)


# speedup vs baseline: 14.3372x; 14.3372x over previous
import functools

import jax
import jax.numpy as jnp
from jax import lax
from jax.experimental import pallas as pl
from jax.experimental.pallas import tpu as pltpu
from jax.experimental.pallas import tpu_sc as plsc

N = 100000
D = 128
C = 400
NC = 2
NS = 16
NW = NC * NS
NUM_CHUNKS = N // C
ROW_UNROLL = 16


def _body(x0_hbm, x1_hbm, wp_hbm, out_hbm, flg_v, x1_v, wt_v, out_v, sem):
  cid = lax.axis_index("c")
  sid = lax.axis_index("s")
  wid = sid * NC + cid

  pltpu.sync_copy(wp_hbm, wt_v)

  def chunk_body(k, carry):
    g = k * NW + wid
    base = g * C
    pltpu.sync_copy(x0_hbm.at[pl.ds(base, C)], flg_v)
    pltpu.sync_copy(x1_hbm.at[pl.ds(base, C)], x1_v)

    lane15 = lax.iota(jnp.int32, 16) == (16 - 1)

    def row_group(j, carry2):
      r0 = j * ROW_UNROLL
      fvec = flg_v[pl.ds(r0, ROW_UNROLL)].astype(jnp.int32)
      xvec = x1_v[pl.ds(r0, ROW_UNROLL)]
      for u in range(ROW_UNROLL):
        r = r0 + u
        fi = fvec[u]
        for c in range(D // 16 - 1):
          out_v[r, pl.ds(c * 16, 16)] = wt_v[fi, pl.ds(c * 16, 16)]
        tail = wt_v[fi, pl.ds(D - 16, 16)]
        xb = jnp.full((16,), xvec[u], jnp.float32)
        out_v[r, pl.ds(D - 16, 16)] = jnp.where(lane15, xb, tail)
      return carry2

    lax.fori_loop(0, C // ROW_UNROLL, row_group, 0)
    pltpu.sync_copy(out_v, out_hbm.at[pl.ds(base, C), :])
    return carry

  nw_chunks = (NUM_CHUNKS - wid + NW - 1) // NW
  lax.fori_loop(0, nw_chunks, chunk_body, 0)


@jax.jit
def kernel(x, W):
  xt = x.T
  x0 = xt[0]
  x1 = xt[1]
  wp = jnp.pad(W, ((0, 0), (0, 1)))
  mesh = plsc.VectorSubcoreMesh(
      core_axis_name="c", subcore_axis_name="s", num_cores=NC,
      num_subcores=NS)
  run = pl.kernel(
      _body,
      out_type=jax.ShapeDtypeStruct((N, D), jnp.float32),
      mesh=mesh,
      compiler_params=pltpu.CompilerParams(
          use_tc_tiling_on_sc=False, needs_layout_passes=False),
      scratch_types=[
          pltpu.VMEM((C,), jnp.float32),
          pltpu.VMEM((C,), jnp.float32),
          pltpu.VMEM((2, D), jnp.float32),
          pltpu.VMEM((C, D), jnp.float32),
          pltpu.SemaphoreType.DMA,
      ],
  )
  return run(x0, x1, wp)


# speedup vs baseline: 15.6801x; 1.0937x over previous
import functools

import jax
import jax.numpy as jnp
from jax import lax
from jax.experimental import pallas as pl
from jax.experimental.pallas import tpu as pltpu
from jax.experimental.pallas import tpu_sc as plsc

N = 100000
D = 128
C = 400
NC = 2
NS = 16
NW = NC * NS
NUM_CHUNKS = N // C
KMAX = (NUM_CHUNKS + NW - 1) // NW
ROW_UNROLL = 16


def _body(x0_hbm, x1_hbm, wp_hbm, out_hbm,
          flg_a, flg_b, x1_a, x1_b, wt_v, out_a, out_b,
          sem_ia, sem_ib, sem_oa, sem_ob):
  cid = lax.axis_index("c")
  sid = lax.axis_index("s")
  wid = sid * NC + cid

  pltpu.sync_copy(wp_hbm, wt_v)

  bufs = ((flg_a, x1_a, out_a, sem_ia, sem_oa),
          (flg_b, x1_b, out_b, sem_ib, sem_ob))
  lane15 = lax.iota(jnp.int32, 16) == (16 - 1)

  def start_in(k):
    f, x, _, si, _ = bufs[k % 2]
    base = (k * NW + wid) * C
    pltpu.async_copy(x0_hbm.at[pl.ds(base, C)], f, si)
    pltpu.async_copy(x1_hbm.at[pl.ds(base, C)], x, si)

  def wait_in(k):
    f, x, _, si, _ = bufs[k % 2]
    pltpu.make_async_copy(x0_hbm.at[pl.ds(0, C)], f, si).wait()
    pltpu.make_async_copy(x1_hbm.at[pl.ds(0, C)], x, si).wait()

  def wait_out(p):
    _, _, o, _, so = bufs[p]
    pltpu.make_async_copy(o, out_hbm.at[pl.ds(0, C), :], so).wait()

  def fill_and_send(k):
    f, x, o, _, so = bufs[k % 2]
    base = (k * NW + wid) * C

    def row_group(j, carry):
      r0 = j * ROW_UNROLL
      fvec = f[pl.ds(r0, ROW_UNROLL)].astype(jnp.int32)
      xvec = x[pl.ds(r0, ROW_UNROLL)]
      for u in range(ROW_UNROLL):
        r = r0 + u
        fi = fvec[u]
        for c in range(D // 16 - 1):
          o[r, pl.ds(c * 16, 16)] = wt_v[fi, pl.ds(c * 16, 16)]
        tail = wt_v[fi, pl.ds(D - 16, 16)]
        xb = jnp.full((16,), xvec[u], jnp.float32)
        o[r, pl.ds(D - 16, 16)] = jnp.where(lane15, xb, tail)
      return carry

    lax.fori_loop(0, C // ROW_UNROLL, row_group, 0)
    pltpu.async_copy(o, out_hbm.at[pl.ds(base, C), :], so)

  start_in(0)
  for k in range(KMAX):
    def step(k=k):
      if k + 1 < KMAX:
        if (k + 1) * NW + NW - 1 < NUM_CHUNKS:
          start_in(k + 1)
        else:
          pl.when((k + 1) * NW + wid < NUM_CHUNKS)(lambda: start_in(k + 1))
      wait_in(k)
      if k >= 2:
        wait_out(k % 2)
      fill_and_send(k)
    if k * NW + NW - 1 < NUM_CHUNKS:
      step()
    else:
      pl.when(k * NW + wid < NUM_CHUNKS)(step)
  wait_out(0)
  wait_out(1)


@jax.jit
def kernel(x, W):
  xt = x.T
  x0 = xt[0]
  x1 = xt[1]
  wp = jnp.pad(W, ((0, 0), (0, 1)))
  mesh = plsc.VectorSubcoreMesh(
      core_axis_name="c", subcore_axis_name="s", num_cores=NC,
      num_subcores=NS)
  run = pl.kernel(
      _body,
      out_type=jax.ShapeDtypeStruct((N, D), jnp.float32),
      mesh=mesh,
      compiler_params=pltpu.CompilerParams(
          use_tc_tiling_on_sc=False, needs_layout_passes=False),
      scratch_types=[
          pltpu.VMEM((C,), jnp.float32),
          pltpu.VMEM((C,), jnp.float32),
          pltpu.VMEM((C,), jnp.float32),
          pltpu.VMEM((C,), jnp.float32),
          pltpu.VMEM((2, D), jnp.float32),
          pltpu.VMEM((C, D), jnp.float32),
          pltpu.VMEM((C, D), jnp.float32),
          pltpu.SemaphoreType.DMA,
          pltpu.SemaphoreType.DMA,
          pltpu.SemaphoreType.DMA,
          pltpu.SemaphoreType.DMA,
      ],
  )
  return run(x0, x1, wp)


# speedup vs baseline: 49.3264x; 3.1458x over previous
import functools

import jax
import jax.numpy as jnp
from jax import lax
from jax.experimental import pallas as pl
from jax.experimental.pallas import tpu as pltpu
from jax.experimental.pallas import tpu_sc as plsc

N = 100000
D = 128
C = 400
NC = 2
NS = 16
NW = NC * NS
NUM_CHUNKS = N // C
KMAX = (NUM_CHUNKS + NW - 1) // NW
ROW_UNROLL = 16


def _body(x0_hbm, x1_hbm, wp_hbm, out_hbm,
          flg_a, flg_b, x1_a, x1_b, wt_v, out_a, out_b,
          sem_ia, sem_ib, sem_oa, sem_ob):
  cid = lax.axis_index("c")
  sid = lax.axis_index("s")
  wid = sid * NC + cid

  pltpu.sync_copy(wp_hbm, wt_v)

  bufs = ((flg_a, x1_a, out_a, sem_ia, sem_oa),
          (flg_b, x1_b, out_b, sem_ib, sem_ob))
  lane15 = lax.iota(jnp.int32, 16) == (16 - 1)
  w0 = [wt_v[0, pl.ds(c * 16, 16)] for c in range(D // 16)]
  w1 = [wt_v[1, pl.ds(c * 16, 16)] for c in range(D // 16)]

  def start_in(k):
    f, x, _, si, _ = bufs[k % 2]
    base = (k * NW + wid) * C
    pltpu.async_copy(x0_hbm.at[pl.ds(base, C)], f, si)
    pltpu.async_copy(x1_hbm.at[pl.ds(base, C)], x, si)

  def wait_in(k):
    f, x, _, si, _ = bufs[k % 2]
    pltpu.make_async_copy(x0_hbm.at[pl.ds(0, C)], f, si).wait()
    pltpu.make_async_copy(x1_hbm.at[pl.ds(0, C)], x, si).wait()

  def wait_out(p):
    _, _, o, _, so = bufs[p]
    pltpu.make_async_copy(o, out_hbm.at[pl.ds(0, C), :], so).wait()

  def fill_and_send(k):
    f, x, o, _, so = bufs[k % 2]
    base = (k * NW + wid) * C

    @functools.partial(plsc.parallel_loop, 0, C // ROW_UNROLL)
    def row_group(j):
      r0 = j * ROW_UNROLL
      fvec = f[pl.ds(r0, ROW_UNROLL)]
      xvec = x[pl.ds(r0, ROW_UNROLL)]
      for u in range(ROW_UNROLL):
        r = r0 + u
        m = jnp.full((16,), fvec[u]) != 0.0
        for c in range(D // 16 - 1):
          o[r, pl.ds(c * 16, 16)] = jnp.where(m, w1[c], w0[c])
        tail = jnp.where(m, w1[D // 16 - 1], w0[D // 16 - 1])
        xb = jnp.full((16,), xvec[u], jnp.float32)
        o[r, pl.ds(D - 16, 16)] = jnp.where(lane15, xb, tail)

    pltpu.async_copy(o, out_hbm.at[pl.ds(base, C), :], so)

  start_in(0)
  for k in range(KMAX):
    def step(k=k):
      if k + 1 < KMAX:
        if (k + 1) * NW + NW - 1 < NUM_CHUNKS:
          start_in(k + 1)
        else:
          pl.when((k + 1) * NW + wid < NUM_CHUNKS)(lambda: start_in(k + 1))
      wait_in(k)
      if k >= 2:
        wait_out(k % 2)
      fill_and_send(k)
    if k * NW + NW - 1 < NUM_CHUNKS:
      step()
    else:
      pl.when(k * NW + wid < NUM_CHUNKS)(step)
  wait_out(0)
  wait_out(1)


@jax.jit
def kernel(x, W):
  xt = x.T
  x0 = xt[0]
  x1 = xt[1]
  wp = jnp.pad(W, ((0, 0), (0, 1)))
  mesh = plsc.VectorSubcoreMesh(
      core_axis_name="c", subcore_axis_name="s", num_cores=NC,
      num_subcores=NS)
  run = pl.kernel(
      _body,
      out_type=jax.ShapeDtypeStruct((N, D), jnp.float32),
      mesh=mesh,
      compiler_params=pltpu.CompilerParams(
          use_tc_tiling_on_sc=False, needs_layout_passes=False),
      scratch_types=[
          pltpu.VMEM((C,), jnp.float32),
          pltpu.VMEM((C,), jnp.float32),
          pltpu.VMEM((C,), jnp.float32),
          pltpu.VMEM((C,), jnp.float32),
          pltpu.VMEM((2, D), jnp.float32),
          pltpu.VMEM((C, D), jnp.float32),
          pltpu.VMEM((C, D), jnp.float32),
          pltpu.SemaphoreType.DMA,
          pltpu.SemaphoreType.DMA,
          pltpu.SemaphoreType.DMA,
          pltpu.SemaphoreType.DMA,
      ],
  )
  return run(x0, x1, wp)
